# initial kernel scaffold (unmeasured)
import jax
import jax.numpy as jnp
from jax import lax
from jax.experimental import pallas as pl
from jax.experimental.pallas import tpu as pltpu

N_DEV = 8


def kernel(x, w_mat):
    m, k_per = x.shape
    _, n = w_mat.shape
    assert m % N_DEV == 0
    chunk = m // N_DEV

    def body(x_ref, w_ref, out_ref, recv_buf, send_sem, recv_sem, ready_sem):
        my = lax.axis_index("i")
        left = lax.rem(my - 1 + N_DEV, N_DEV)
        right = lax.rem(my + 1, N_DEV)

        out_ref[:, :] = jnp.dot(
            x_ref[:, :], w_ref[:, :], preferred_element_type=jnp.float32
        )

        for s in range(N_DEV - 1):
            send_c = lax.rem(my - s + N_DEV, N_DEV)
            recv_c = lax.rem(my - s - 1 + 2 * N_DEV, N_DEV)
            pl.semaphore_signal(
                ready_sem, inc=1,
                device_id=(left,), device_id_type=pl.DeviceIdType.MESH,
            )
            pl.semaphore_wait(ready_sem, 1)
            rdma = pltpu.make_async_remote_copy(
                src_ref=out_ref.at[pl.ds(send_c * chunk, chunk), :],
                dst_ref=recv_buf,
                send_sem=send_sem,
                recv_sem=recv_sem,
                device_id=(right,),
                device_id_type=pl.DeviceIdType.MESH,
            )
            rdma.start()
            rdma.wait()
            idx = pl.ds(recv_c * chunk, chunk)
            out_ref[idx, :] = out_ref[idx, :] + recv_buf[:, :]

        for s in range(N_DEV - 1):
            send_c = lax.rem(my + 1 - s + 2 * N_DEV, N_DEV)
            pl.semaphore_signal(
                ready_sem, inc=1,
                device_id=(left,), device_id_type=pl.DeviceIdType.MESH,
            )
            pl.semaphore_wait(ready_sem, 1)
            rdma = pltpu.make_async_remote_copy(
                src_ref=out_ref.at[pl.ds(send_c * chunk, chunk), :],
                dst_ref=out_ref.at[pl.ds(send_c * chunk, chunk), :],
                send_sem=send_sem,
                recv_sem=recv_sem,
                device_id=(right,),
                device_id_type=pl.DeviceIdType.MESH,
            )
            rdma.start()
            rdma.wait()

    return pl.pallas_call(
        body,
        out_shape=jax.ShapeDtypeStruct((m, n), jnp.float32),
        in_specs=[
            pl.BlockSpec(memory_space=pltpu.VMEM),
            pl.BlockSpec(memory_space=pltpu.VMEM),
        ],
        out_specs=pl.BlockSpec(memory_space=pltpu.VMEM),
        scratch_shapes=[
            pltpu.VMEM((chunk, n), jnp.float32),
            pltpu.SemaphoreType.DMA,
            pltpu.SemaphoreType.DMA,
            pltpu.SemaphoreType.REGULAR,
        ],
        compiler_params=pltpu.CompilerParams(collective_id=0),
    )(x, w_mat)


# baseline (device time: 733198 ns/iter reference)
import jax
import jax.numpy as jnp
from jax import lax
from jax.experimental import pallas as pl
from jax.experimental.pallas import tpu as pltpu

N_DEV = 8


def kernel(x, w_mat):
    m, k_per = x.shape
    _, n = w_mat.shape
    assert m % N_DEV == 0
    chunk = m // N_DEV

    def body(x_ref, w_ref, out_ref, recv_buf, send_sem, recv_sem, ready_sem):
        my = lax.axis_index("i")
        left = lax.rem(my - 1 + N_DEV, N_DEV)
        right = lax.rem(my + 1, N_DEV)

        out_ref[:, :] = jnp.dot(
            x_ref[:, :], w_ref[:, :], preferred_element_type=jnp.float32
        )

        for s in range(N_DEV - 1):
            send_c = lax.rem(my - s + N_DEV, N_DEV)
            recv_c = lax.rem(my - s - 1 + 2 * N_DEV, N_DEV)
            pl.semaphore_signal(
                ready_sem, inc=1,
                device_id=(left,), device_id_type=pl.DeviceIdType.MESH,
            )
            pl.semaphore_wait(ready_sem, 1)
            rdma = pltpu.make_async_remote_copy(
                src_ref=out_ref.at[pl.ds(send_c * chunk, chunk), :],
                dst_ref=recv_buf,
                send_sem=send_sem,
                recv_sem=recv_sem,
                device_id=(right,),
                device_id_type=pl.DeviceIdType.MESH,
            )
            rdma.start()
            rdma.wait()
            idx = pl.ds(recv_c * chunk, chunk)
            out_ref[idx, :] = out_ref[idx, :] + recv_buf[:, :]

        for s in range(N_DEV - 1):
            send_c = lax.rem(my + 1 - s + 2 * N_DEV, N_DEV)
            pl.semaphore_signal(
                ready_sem, inc=1,
                device_id=(left,), device_id_type=pl.DeviceIdType.MESH,
            )
            pl.semaphore_wait(ready_sem, 1)
            rdma = pltpu.make_async_remote_copy(
                src_ref=out_ref.at[pl.ds(send_c * chunk, chunk), :],
                dst_ref=out_ref.at[pl.ds(send_c * chunk, chunk), :],
                send_sem=send_sem,
                recv_sem=recv_sem,
                device_id=(right,),
                device_id_type=pl.DeviceIdType.MESH,
            )
            rdma.start()
            rdma.wait()

    return pl.pallas_call(
        body,
        out_shape=jax.ShapeDtypeStruct((m, n), jnp.float32),
        in_specs=[
            pl.BlockSpec(memory_space=pltpu.VMEM),
            pl.BlockSpec(memory_space=pltpu.VMEM),
        ],
        out_specs=pl.BlockSpec(memory_space=pltpu.VMEM),
        scratch_shapes=[
            pltpu.VMEM((chunk, n), jnp.float32),
            pltpu.SemaphoreType.DMA,
            pltpu.SemaphoreType.DMA,
            pltpu.SemaphoreType.REGULAR,
        ],
        compiler_params=pltpu.CompilerParams(
            vmem_limit_bytes=100 * 1024 * 1024,
        ),
    )(x, w_mat)


# device time: 419612 ns/iter; 1.7473x vs baseline; 1.7473x over previous
import jax
import jax.numpy as jnp
from jax import lax
from jax.experimental import pallas as pl
from jax.experimental.pallas import tpu as pltpu

N_DEV = 8


def kernel(x, w_mat):
    m, k_per = x.shape
    _, n = w_mat.shape
    halfm = m // 2
    assert halfm % N_DEV == 0
    hchunk = halfm // N_DEV

    def body(
        x_ref, w_ref, out_ref,
        recv_a, recv_b,
        send_sem_a, recv_sem_a, send_sem_b, recv_sem_b,
        ready_a, ready_b,
    ):
        my = lax.axis_index("i")
        left = lax.rem(my - 1 + N_DEV, N_DEV)
        right = lax.rem(my + 1, N_DEV)

        def rows_a(c):
            return pl.ds(c * hchunk, hchunk)

        def rows_b(c):
            return pl.ds(halfm + c * hchunk, hchunk)

        out_ref[:, :] = jnp.dot(
            x_ref[:, :], w_ref[:, :], preferred_element_type=jnp.float32
        )

        def exchange(send_a_c, send_b_c, into_out_a=None, into_out_b=None):
            pl.semaphore_signal(
                ready_a, inc=1,
                device_id=(left,), device_id_type=pl.DeviceIdType.MESH,
            )
            pl.semaphore_signal(
                ready_b, inc=1,
                device_id=(right,), device_id_type=pl.DeviceIdType.MESH,
            )
            pl.semaphore_wait(ready_a, 1)
            pl.semaphore_wait(ready_b, 1)
            dst_a = recv_a if into_out_a is None else out_ref.at[rows_a(into_out_a), :]
            dst_b = recv_b if into_out_b is None else out_ref.at[rows_b(into_out_b), :]
            rdma_a = pltpu.make_async_remote_copy(
                src_ref=out_ref.at[rows_a(send_a_c), :],
                dst_ref=dst_a,
                send_sem=send_sem_a,
                recv_sem=recv_sem_a,
                device_id=(right,),
                device_id_type=pl.DeviceIdType.MESH,
            )
            rdma_b = pltpu.make_async_remote_copy(
                src_ref=out_ref.at[rows_b(send_b_c), :],
                dst_ref=dst_b,
                send_sem=send_sem_b,
                recv_sem=recv_sem_b,
                device_id=(left,),
                device_id_type=pl.DeviceIdType.MESH,
            )
            rdma_a.start()
            rdma_b.start()
            rdma_a.wait()
            rdma_b.wait()

        for s in range(N_DEV - 1):
            send_a_c = lax.rem(my - s + N_DEV, N_DEV)
            recv_a_c = lax.rem(my - s - 1 + 2 * N_DEV, N_DEV)
            send_b_c = lax.rem(my + s, N_DEV)
            recv_b_c = lax.rem(my + s + 1, N_DEV)
            exchange(send_a_c, send_b_c)
            ia = rows_a(recv_a_c)
            out_ref[ia, :] = out_ref[ia, :] + recv_a[:, :]
            ib = rows_b(recv_b_c)
            out_ref[ib, :] = out_ref[ib, :] + recv_b[:, :]

        for s in range(N_DEV - 1):
            send_a_c = lax.rem(my + 1 - s + 2 * N_DEV, N_DEV)
            send_b_c = lax.rem(my - 1 + s + N_DEV, N_DEV)
            exchange(send_a_c, send_b_c,
                     into_out_a=send_a_c, into_out_b=send_b_c)

    return pl.pallas_call(
        body,
        out_shape=jax.ShapeDtypeStruct((m, n), jnp.float32),
        in_specs=[
            pl.BlockSpec(memory_space=pltpu.VMEM),
            pl.BlockSpec(memory_space=pltpu.VMEM),
        ],
        out_specs=pl.BlockSpec(memory_space=pltpu.VMEM),
        scratch_shapes=[
            pltpu.VMEM((hchunk, n), jnp.float32),
            pltpu.VMEM((hchunk, n), jnp.float32),
            pltpu.SemaphoreType.DMA,
            pltpu.SemaphoreType.DMA,
            pltpu.SemaphoreType.DMA,
            pltpu.SemaphoreType.DMA,
            pltpu.SemaphoreType.REGULAR,
            pltpu.SemaphoreType.REGULAR,
        ],
        compiler_params=pltpu.CompilerParams(
            vmem_limit_bytes=100 * 1024 * 1024,
        ),
    )(x, w_mat)


# device time: 404100 ns/iter; 1.8144x vs baseline; 1.0384x over previous
import jax
import jax.numpy as jnp
from jax import lax
from jax.experimental import pallas as pl
from jax.experimental.pallas import tpu as pltpu

N_DEV = 8
N_STEP = 2 * (N_DEV - 1)


def kernel(x, w_mat):
    m, k_per = x.shape
    _, n = w_mat.shape
    halfm = m // 2
    assert halfm % N_DEV == 0
    hchunk = halfm // N_DEV

    def body(
        x_ref, w_ref, out_ref,
        recv_a, recv_b,
        send_sem_a, recv_sem_a, send_sem_b, recv_sem_b,
        ready_a, ready_b,
    ):
        my = lax.axis_index("i")
        left = lax.rem(my - 1 + N_DEV, N_DEV)
        right = lax.rem(my + 1, N_DEV)

        def rows_a(c):
            return pl.ds(c * hchunk, hchunk)

        def rows_b(c):
            return pl.ds(halfm + c * hchunk, hchunk)

        out_ref[:, :] = jnp.dot(
            x_ref[:, :], w_ref[:, :], preferred_element_type=jnp.float32
        )

        for _ in range(2):
            pl.semaphore_signal(
                ready_a, inc=1,
                device_id=(left,), device_id_type=pl.DeviceIdType.MESH,
            )
            pl.semaphore_signal(
                ready_b, inc=1,
                device_id=(right,), device_id_type=pl.DeviceIdType.MESH,
            )

        def start_pair(s, send_a_c, send_b_c, ag):
            slot = s % 2
            pl.semaphore_wait(ready_a, 1)
            pl.semaphore_wait(ready_b, 1)
            dst_a = recv_a.at[slot] if not ag else out_ref.at[rows_a(send_a_c), :]
            dst_b = recv_b.at[slot] if not ag else out_ref.at[rows_b(send_b_c), :]
            rdma_a = pltpu.make_async_remote_copy(
                src_ref=out_ref.at[rows_a(send_a_c), :],
                dst_ref=dst_a,
                send_sem=send_sem_a.at[slot],
                recv_sem=recv_sem_a.at[slot],
                device_id=(right,),
                device_id_type=pl.DeviceIdType.MESH,
            )
            rdma_b = pltpu.make_async_remote_copy(
                src_ref=out_ref.at[rows_b(send_b_c), :],
                dst_ref=dst_b,
                send_sem=send_sem_b.at[slot],
                recv_sem=recv_sem_b.at[slot],
                device_id=(left,),
                device_id_type=pl.DeviceIdType.MESH,
            )
            rdma_a.start()
            rdma_b.start()
            return rdma_a, rdma_b

        def finish_recv(s, rdma_a, rdma_b, recv_a_c, recv_b_c, ag):
            slot = s % 2
            rdma_a.wait_recv()
            if not ag:
                ia = rows_a(recv_a_c)
                out_ref[ia, :] = out_ref[ia, :] + recv_a[slot, :, :]
            rdma_b.wait_recv()
            if not ag:
                ib = rows_b(recv_b_c)
                out_ref[ib, :] = out_ref[ib, :] + recv_b[slot, :, :]
            if s < N_STEP - 2:
                pl.semaphore_signal(
                    ready_a, inc=1,
                    device_id=(left,), device_id_type=pl.DeviceIdType.MESH,
                )
                pl.semaphore_signal(
                    ready_b, inc=1,
                    device_id=(right,), device_id_type=pl.DeviceIdType.MESH,
                )

        prev = {}

        def chunk_ids(s):
            if s < N_DEV - 1:
                return (
                    lax.rem(my - s + N_DEV, N_DEV),
                    lax.rem(my - s - 1 + 2 * N_DEV, N_DEV),
                    lax.rem(my + s, N_DEV),
                    lax.rem(my + s + 1, N_DEV),
                )
            t = s - (N_DEV - 1)
            return (
                lax.rem(my + 1 - t + 2 * N_DEV, N_DEV),
                lax.rem(my - t + 2 * N_DEV, N_DEV),
                lax.rem(my - 1 + t + N_DEV, N_DEV),
                lax.rem(my + t, N_DEV),
            )

        for s in range(N_STEP):
            ag = s >= N_DEV - 1
            send_a_c, recv_a_c, send_b_c, recv_b_c = chunk_ids(s)
            if s >= 2:
                pa, pb = prev.pop(s - 2)
                pa.wait_send()
                pb.wait_send()
            rdma_a, rdma_b = start_pair(s, send_a_c, send_b_c, ag)
            prev[s] = (rdma_a, rdma_b)
            finish_recv(s, rdma_a, rdma_b, recv_a_c, recv_b_c, ag)

        for s in (N_STEP - 2, N_STEP - 1):
            pa, pb = prev.pop(s)
            pa.wait_send()
            pb.wait_send()

    return pl.pallas_call(
        body,
        out_shape=jax.ShapeDtypeStruct((m, n), jnp.float32),
        in_specs=[
            pl.BlockSpec(memory_space=pltpu.VMEM),
            pl.BlockSpec(memory_space=pltpu.VMEM),
        ],
        out_specs=pl.BlockSpec(memory_space=pltpu.VMEM),
        scratch_shapes=[
            pltpu.VMEM((2, hchunk, n), jnp.float32),
            pltpu.VMEM((2, hchunk, n), jnp.float32),
            pltpu.SemaphoreType.DMA((2,)),
            pltpu.SemaphoreType.DMA((2,)),
            pltpu.SemaphoreType.DMA((2,)),
            pltpu.SemaphoreType.DMA((2,)),
            pltpu.SemaphoreType.REGULAR,
            pltpu.SemaphoreType.REGULAR,
        ],
        compiler_params=pltpu.CompilerParams(
            vmem_limit_bytes=100 * 1024 * 1024,
        ),
    )(x, w_mat)


# device time: 364505 ns/iter; 2.0115x vs baseline; 1.1086x over previous
import jax
import jax.numpy as jnp
from jax import lax
from jax.experimental import pallas as pl
from jax.experimental.pallas import tpu as pltpu

N_DEV = 8
N_STEP = 2 * (N_DEV - 1)
K_SUB = 2


def kernel(x, w_mat):
    m, k_per = x.shape
    _, n = w_mat.shape
    halfm = m // 2
    assert halfm % N_DEV == 0
    hchunk = halfm // N_DEV
    assert hchunk % K_SUB == 0
    sub = hchunk // K_SUB

    def body(
        x_ref, w_ref, out_ref,
        recv_a, recv_b,
        send_sem_a, recv_sem_a, send_sem_b, recv_sem_b,
        ready_a, ready_b,
    ):
        my = lax.axis_index("i")
        left = lax.rem(my - 1 + N_DEV, N_DEV)
        right = lax.rem(my + 1, N_DEV)

        def rows_a(c, k):
            return pl.ds(c * hchunk + k * sub, sub)

        def rows_b(c, k):
            return pl.ds(halfm + c * hchunk + k * sub, sub)

        def gemm_chunk(c, pred):
            @pl.when(pred)
            def _():
                ia = pl.ds(c * hchunk, hchunk)
                out_ref[ia, :] = jnp.dot(
                    x_ref[ia, :], w_ref[:, :],
                    preferred_element_type=jnp.float32,
                )
                ib = pl.ds(halfm + c * hchunk, hchunk)
                out_ref[ib, :] = jnp.dot(
                    x_ref[ib, :], w_ref[:, :],
                    preferred_element_type=jnp.float32,
                )

        def chunk_ids(s):
            if s < N_DEV - 1:
                return (
                    lax.rem(my - s + N_DEV, N_DEV),
                    lax.rem(my - s - 1 + 2 * N_DEV, N_DEV),
                    lax.rem(my + s, N_DEV),
                    lax.rem(my + s + 1, N_DEV),
                )
            t = s - (N_DEV - 1)
            return (
                lax.rem(my + 1 - t + 2 * N_DEV, N_DEV),
                lax.rem(my - t + 2 * N_DEV, N_DEV),
                lax.rem(my - 1 + t + N_DEV, N_DEV),
                lax.rem(my + t, N_DEV),
            )

        def issue(s, k):
            slot = s % 2
            ag = s >= N_DEV - 1
            send_a_c, _, send_b_c, _ = chunk_ids(s)
            dst_a = recv_a.at[slot, k] if not ag else out_ref.at[rows_a(send_a_c, k), :]
            dst_b = recv_b.at[slot, k] if not ag else out_ref.at[rows_b(send_b_c, k), :]
            rdma_a = pltpu.make_async_remote_copy(
                src_ref=out_ref.at[rows_a(send_a_c, k), :],
                dst_ref=dst_a,
                send_sem=send_sem_a.at[slot, k],
                recv_sem=recv_sem_a.at[slot, k],
                device_id=(right,),
                device_id_type=pl.DeviceIdType.MESH,
            )
            rdma_b = pltpu.make_async_remote_copy(
                src_ref=out_ref.at[rows_b(send_b_c, k), :],
                dst_ref=dst_b,
                send_sem=send_sem_b.at[slot, k],
                recv_sem=recv_sem_b.at[slot, k],
                device_id=(left,),
                device_id_type=pl.DeviceIdType.MESH,
            )
            rdma_a.start()
            rdma_b.start()
            return rdma_a, rdma_b

        def grant(inc=1):
            pl.semaphore_signal(
                ready_a, inc=inc,
                device_id=(left,), device_id_type=pl.DeviceIdType.MESH,
            )
            pl.semaphore_signal(
                ready_b, inc=inc,
                device_id=(right,), device_id_type=pl.DeviceIdType.MESH,
            )

        grant(inc=2)

        gemm_chunk(my, True)
        pl.semaphore_wait(ready_a, 1)
        pl.semaphore_wait(ready_b, 1)
        inflight = {(0, k): issue(0, k) for k in range(K_SUB)}
        for c in range(N_DEV):
            gemm_chunk(c, c != my)

        for s in range(N_STEP):
            ag = s >= N_DEV - 1
            _, recv_a_c, _, recv_b_c = chunk_ids(s)
            if s + 1 < N_STEP:
                pl.semaphore_wait(ready_a, 1)
                pl.semaphore_wait(ready_b, 1)
            for k in range(K_SUB):
                rdma_a, rdma_b = inflight[(s, k)]
                rdma_a.wait_recv()
                if not ag:
                    ia = rows_a(recv_a_c, k)
                    out_ref[ia, :] = out_ref[ia, :] + recv_a[s % 2, k, :, :]
                rdma_b.wait_recv()
                if not ag:
                    ib = rows_b(recv_b_c, k)
                    out_ref[ib, :] = out_ref[ib, :] + recv_b[s % 2, k, :, :]
                if s + 1 < N_STEP:
                    if s >= 1:
                        pa, pb = inflight.pop((s - 1, k))
                        pa.wait_send()
                        pb.wait_send()
                    inflight[(s + 1, k)] = issue(s + 1, k)
            if s < N_STEP - 2:
                grant()

        for key in sorted(inflight):
            pa, pb = inflight.pop(key)
            pa.wait_send()
            pb.wait_send()

    return pl.pallas_call(
        body,
        out_shape=jax.ShapeDtypeStruct((m, n), jnp.float32),
        in_specs=[
            pl.BlockSpec(memory_space=pltpu.VMEM),
            pl.BlockSpec(memory_space=pltpu.VMEM),
        ],
        out_specs=pl.BlockSpec(memory_space=pltpu.VMEM),
        scratch_shapes=[
            pltpu.VMEM((2, K_SUB, sub, n), jnp.float32),
            pltpu.VMEM((2, K_SUB, sub, n), jnp.float32),
            pltpu.SemaphoreType.DMA((2, K_SUB)),
            pltpu.SemaphoreType.DMA((2, K_SUB)),
            pltpu.SemaphoreType.DMA((2, K_SUB)),
            pltpu.SemaphoreType.DMA((2, K_SUB)),
            pltpu.SemaphoreType.REGULAR,
            pltpu.SemaphoreType.REGULAR,
        ],
        compiler_params=pltpu.CompilerParams(
            vmem_limit_bytes=100 * 1024 * 1024,
        ),
    )(x, w_mat)
